# Initial kernel scaffold; baseline (speedup 1.0000x reference)
import jax
import jax.numpy as jnp
from jax.experimental import pallas as pl


def kernel(h_flat, z_flat, e_feat, path_j, path_k, path_r0j, path_r0k, path_rjk, path_cosangle, path_batch, bsz, params):
    raise NotImplementedError("write your pallas kernel here")


# speedup vs baseline: 8.1604x; 8.1604x over previous
import functools

import jax
import jax.numpy as jnp
from jax import lax
from jax.experimental import pallas as pl
from jax.experimental.pallas import tpu as pltpu
from jax.experimental.pallas import tpu_sc as plsc

_CUTOFF = 6.0
_B = 64
_BP = 1024
_CH = 128


def _sc_embed(zemb, zf):
    n_nodes = zf.shape[0]
    z_dim = zemb.shape[1]
    info = plsc.get_sparse_core_info()
    nc, ns = info.num_cores, info.num_subcores
    nw = nc * ns
    bpw = n_nodes // nw
    ch = 64
    nch = bpw // ch
    mesh = plsc.VectorSubcoreMesh(core_axis_name="c", subcore_axis_name="s")

    @functools.partial(
        pl.kernel,
        mesh=mesh,
        out_type=jax.ShapeDtypeStruct((n_nodes, z_dim), jnp.float32),
        scratch_types=[
            pltpu.VMEM((bpw,), jnp.int32),
            pltpu.VMEM((ch, z_dim), jnp.float32),
            pltpu.SemaphoreType.DMA,
        ],
    )
    def embed_kernel(zemb_hbm, zf_hbm, emb_out, zf_v, rows_v, sem):
        wid = lax.axis_index("s") * nc + lax.axis_index("c")
        base = wid * bpw
        pltpu.sync_copy(zf_hbm.at[pl.ds(base, bpw)], zf_v)
        for c in range(nch):
            off = c * ch
            pltpu.async_copy(
                zemb_hbm.at[zf_v.at[pl.ds(off, ch)]], rows_v, sem).wait()
            pltpu.sync_copy(rows_v, emb_out.at[pl.ds(base + off, ch)])

    return embed_kernel(zemb, zf)


def _sc_gather(h, zemb, zf, pj, pk):
    n_nodes, atom_dim = h.shape
    z_dim = zemb.shape[1]
    pp = pj.shape[0]
    emb = _sc_embed(zemb, zf)
    info = plsc.get_sparse_core_info()
    nc, ns = info.num_cores, info.num_subcores
    nw = nc * ns
    bpw = pp // nw
    nch = bpw // _CH
    mesh = plsc.VectorSubcoreMesh(core_axis_name="c", subcore_axis_name="s")

    @functools.partial(
        pl.kernel,
        mesh=mesh,
        out_type=(
            jax.ShapeDtypeStruct((pp, atom_dim), jnp.float32),
            jax.ShapeDtypeStruct((pp, atom_dim), jnp.float32),
            jax.ShapeDtypeStruct((pp, z_dim), jnp.float32),
            jax.ShapeDtypeStruct((pp, z_dim), jnp.float32),
        ),
        scratch_types=[
            pltpu.VMEM((bpw,), jnp.int32),
            pltpu.VMEM((bpw,), jnp.int32),
            pltpu.VMEM((_CH, atom_dim), jnp.float32),
            pltpu.VMEM((_CH, atom_dim), jnp.float32),
            pltpu.VMEM((_CH, z_dim), jnp.float32),
            pltpu.VMEM((_CH, z_dim), jnp.float32),
            pltpu.SemaphoreType.DMA,
            pltpu.SemaphoreType.DMA,
            pltpu.SemaphoreType.DMA,
            pltpu.SemaphoreType.DMA,
        ],
    )
    def gather_kernel(h_hbm, emb_hbm, pj_hbm, pk_hbm,
                      hj_out, hk_out, ej_out, ek_out,
                      pj_v, pk_v, hj_v, hk_v, ej_v, ek_v,
                      sem_hj, sem_hk, sem_ej, sem_ek):
        wid = lax.axis_index("s") * nc + lax.axis_index("c")
        base = wid * bpw
        pltpu.sync_copy(pj_hbm.at[pl.ds(base, bpw)], pj_v)
        pltpu.sync_copy(pk_hbm.at[pl.ds(base, bpw)], pk_v)
        for c in range(nch):
            off = c * _CH
            cp_hj = pltpu.async_copy(h_hbm.at[pj_v.at[pl.ds(off, _CH)]], hj_v, sem_hj)
            cp_hk = pltpu.async_copy(h_hbm.at[pk_v.at[pl.ds(off, _CH)]], hk_v, sem_hk)
            cp_ej = pltpu.async_copy(emb_hbm.at[pj_v.at[pl.ds(off, _CH)]], ej_v, sem_ej)
            cp_ek = pltpu.async_copy(emb_hbm.at[pk_v.at[pl.ds(off, _CH)]], ek_v, sem_ek)
            cp_hj.wait()
            pltpu.sync_copy(hj_v, hj_out.at[pl.ds(base + off, _CH)])
            cp_hk.wait()
            pltpu.sync_copy(hk_v, hk_out.at[pl.ds(base + off, _CH)])
            cp_ej.wait()
            pltpu.sync_copy(ej_v, ej_out.at[pl.ds(base + off, _CH)])
            cp_ek.wait()
            pltpu.sync_copy(ek_v, ek_out.at[pl.ds(base + off, _CH)])

    return gather_kernel(h, emb, pj, pk)


def _make_main_body(bp, b, n_en, s_dim, rbf_dim, cutoff, z_dim):
    f32 = jnp.float32
    width = cutoff / (rbf_dim - 1)

    def body(hj_ref, hk_ref, ej_ref, ek_ref, r0j_ref, r0k_ref, rjk_ref,
             ca_ref, seg_ref,
             wg1hj_ref, wg1hk_ref, wg1a_ref, wg1b_ref, wg1c_ref, wg1cos_ref,
             bg1_ref, wg2_ref, bg2_ref, wg3_ref, bg3_ref,
             ef_ref, w1pj_ref, w1pk_ref, w1pe_ref, b1p_ref,
             w2p_ref, b2p_ref, w3p_ref, b3p_ref,
             agg_ref, norm_ref):
        @pl.when(pl.program_id(0) == 0)
        def _init():
            agg_ref[...] = jnp.zeros_like(agg_ref)
            norm_ref[...] = jnp.zeros_like(norm_ref)

        cent = lax.broadcasted_iota(
            jnp.int32, (1, rbf_dim), 1).astype(f32) * width

        def dot(a, bb):
            return jnp.dot(a, bb, preferred_element_type=f32)

        def rbf(rc):
            rr = jnp.minimum(rc, cutoff)
            return jnp.exp(-0.5 * ((rr - cent) / width) ** 2)

        def ccut(rc):
            return 0.5 * (jnp.cos(jnp.pi * rc / cutoff) + 1.0) * (
                rc < cutoff).astype(f32)

        r0j = r0j_ref[...]
        r0k = r0k_ref[...]
        rjk = rjk_ref[...]
        x = (dot(hj_ref[...], wg1hj_ref[...])
             + dot(hk_ref[...], wg1hk_ref[...])
             + dot(rbf(r0j), wg1a_ref[...])
             + dot(rbf(r0k), wg1b_ref[...])
             + dot(rbf(rjk), wg1c_ref[...])
             + ca_ref[...] * wg1cos_ref[...]
             + bg1_ref[...])
        x = jax.nn.silu(x)
        x = jax.nn.silu(dot(x, wg2_ref[...]) + bg2_ref[...])
        gg = dot(x, wg3_ref[...]) + bg3_ref[...]

        cw = ccut(r0j) * ccut(r0k) * ccut(rjk)
        ohw = jnp.where(
            seg_ref[...] == lax.broadcasted_iota(jnp.int32, (bp, b), 1),
            cw, 0.0)

        epre = dot(ef_ref[...], w1pe_ref[...]) + b1p_ref[...]
        gp = (dot(ej_ref[...][:, :z_dim], w1pj_ref[...])
              + dot(ek_ref[...][:, :z_dim], w1pk_ref[...]))
        pre = jnp.concatenate(
            [gp + epre[e:e + 1, :] for e in range(n_en)], axis=0)
        h1 = jax.nn.silu(pre)
        h2 = jax.nn.silu(dot(h1, w2p_ref[...]) + b2p_ref[...])
        ge = dot(h2, w3p_ref[...]) + b3p_ref[...]

        tdims = (((0,), (0,)), ((), ()))
        for e in range(n_en):
            me = ge[e * bp:(e + 1) * bp, :] * gg
            agg_ref[:, e * s_dim:(e + 1) * s_dim] += lax.dot_general(
                ohw, me, tdims, preferred_element_type=f32)
        norm_ref[...] += jnp.sum(ohw, axis=0, keepdims=True)

    return body


def _out_body(agg_ref, nrm_ref, wo1_ref, bo1_ref, wo2_ref, bo2_ref, out_ref):
    x = agg_ref[...] / jnp.maximum(nrm_ref[...], 1e-8)
    h = jax.nn.silu(
        jnp.dot(x, wo1_ref[...], preferred_element_type=jnp.float32)
        + bo1_ref[...])
    out_ref[...] = (
        jnp.dot(h, wo2_ref[...], preferred_element_type=jnp.float32)
        + bo2_ref[...])


def _main_call(args, bp, b, n_en, s_dim, rbf_dim, z_dim, interp_shapes):
    pp, atom_dim = interp_shapes["hj"]
    ze_pad = interp_shapes["ej"][1]
    nblk = pp // bp
    row_spec = lambda d: pl.BlockSpec((bp, d), lambda i: (i, 0))
    full = lambda shape: pl.BlockSpec(shape, lambda i: (0, 0))
    in_specs = [
        row_spec(atom_dim), row_spec(atom_dim), row_spec(ze_pad),
        row_spec(ze_pad), row_spec(1), row_spec(1), row_spec(1), row_spec(1),
        row_spec(1),
    ] + [full(a.shape) for a in args[9:]]
    out_specs = [
        pl.BlockSpec((b, n_en * s_dim), lambda i: (0, 0)),
        pl.BlockSpec((1, b), lambda i: (0, 0)),
    ]
    out_shape = [
        jax.ShapeDtypeStruct((b, n_en * s_dim), jnp.float32),
        jax.ShapeDtypeStruct((1, b), jnp.float32),
    ]
    return pl.pallas_call(
        _make_main_body(bp, b, n_en, s_dim, rbf_dim, _CUTOFF, z_dim),
        grid=(nblk,),
        in_specs=in_specs,
        out_specs=out_specs,
        out_shape=out_shape,
    )(*args)


def kernel(h_flat, z_flat, e_feat, path_j, path_k, path_r0j, path_r0k,
           path_rjk, path_cosangle, path_batch, bsz, params):
    f32 = jnp.float32
    i32 = jnp.int32
    h_flat = h_flat.astype(f32)
    e_feat = e_feat.astype(f32)
    zemb = params["z_emb"].astype(f32)
    atom_dim = h_flat.shape[1]
    z_dim = zemb.shape[1]
    n_en = e_feat.shape[0]

    (g_w1, g_b1), (g_w2, g_b2), (g_w3, g_b3) = params["geom_mlp"]
    (p_w1, p_b1), (p_w2, p_b2), (p_w3, p_b3) = params["pair_mlp"]
    (o_w1, o_b1), (o_w2, o_b2) = params["out_mlp"]
    s_dim = g_w3.shape[1]
    rbf_dim = (g_w1.shape[0] - 2 * atom_dim - 1) // 3
    out_dim = o_w2.shape[1]

    p = path_j.shape[0]
    p_pad = ((p + 4095) // 4096) * 4096
    pad = p_pad - p
    pj = jnp.pad(path_j.astype(i32), (0, pad))
    pk = jnp.pad(path_k.astype(i32), (0, pad))
    seg = jnp.pad(jnp.minimum(path_batch, bsz - 1).astype(i32), (0, pad))
    r0j = jnp.pad(path_r0j.astype(f32), (0, pad), constant_values=1e9)
    r0k = jnp.pad(path_r0k.astype(f32), (0, pad), constant_values=1e9)
    rjk = jnp.pad(path_rjk.astype(f32), (0, pad), constant_values=1e9)
    ca = jnp.pad(path_cosangle.astype(f32), (0, pad))

    zemb_pad = jnp.pad(zemb, ((0, 0), (0, 128 - z_dim)))
    hj, hk, ej, ek = _sc_gather(h_flat, zemb_pad, z_flat.astype(i32), pj, pk)

    a = atom_dim
    r = rbf_dim
    wg1hj = g_w1[0:a]
    wg1hk = g_w1[a:2 * a]
    wg1a = g_w1[2 * a:2 * a + r]
    wg1b = g_w1[2 * a + r:2 * a + 2 * r]
    wg1c = g_w1[2 * a + 2 * r:2 * a + 3 * r]
    wg1cos = g_w1[2 * a + 3 * r:2 * a + 3 * r + 1]
    w1pj = p_w1[0:z_dim]
    w1pk = p_w1[z_dim:2 * z_dim]
    w1pe = p_w1[2 * z_dim:2 * z_dim + e_feat.shape[1]]

    args = (
        hj, hk, ej, ek,
        r0j.reshape(p_pad, 1), r0k.reshape(p_pad, 1), rjk.reshape(p_pad, 1),
        ca.reshape(p_pad, 1), seg.reshape(p_pad, 1),
        wg1hj, wg1hk, wg1a, wg1b, wg1c, wg1cos, g_b1.reshape(1, -1),
        g_w2, g_b2.reshape(1, -1), g_w3, g_b3.reshape(1, -1),
        e_feat, w1pj, w1pk, w1pe, p_b1.reshape(1, -1),
        p_w2, p_b2.reshape(1, -1), p_w3, p_b3.reshape(1, -1),
    )
    agg, normv = _main_call(
        args, _BP, _B, n_en, s_dim, rbf_dim, z_dim,
        {"hj": hj.shape, "ej": ej.shape})

    aggr = agg.reshape(_B, n_en, s_dim).reshape(_B * n_en, s_dim)
    nrep = jnp.broadcast_to(
        normv.reshape(_B, 1), (_B, n_en)).reshape(_B * n_en, 1)
    out2 = pl.pallas_call(
        _out_body,
        out_shape=jax.ShapeDtypeStruct((_B * n_en, out_dim), jnp.float32),
    )(aggr, nrep, o_w1, o_b1.reshape(1, -1), o_w2, o_b2.reshape(1, -1))
    return out2.reshape(_B, n_en, out_dim)


# speedup vs baseline: 8.3569x; 1.0241x over previous
import functools

import jax
import jax.numpy as jnp
from jax import lax
from jax.experimental import pallas as pl
from jax.experimental.pallas import tpu as pltpu
from jax.experimental.pallas import tpu_sc as plsc

_CUTOFF = 6.0
_B = 64
_BP = 1024
_CH = 128


def _sc_embed(zemb, zf):
    n_nodes = zf.shape[0]
    z_dim = zemb.shape[1]
    info = plsc.get_sparse_core_info()
    nc, ns = info.num_cores, info.num_subcores
    nw = nc * ns
    bpw = n_nodes // nw
    ch = 64
    nch = bpw // ch
    mesh = plsc.VectorSubcoreMesh(core_axis_name="c", subcore_axis_name="s")

    @functools.partial(
        pl.kernel,
        mesh=mesh,
        out_type=jax.ShapeDtypeStruct((n_nodes, z_dim), jnp.float32),
        scratch_types=[
            pltpu.VMEM((bpw,), jnp.int32),
            pltpu.VMEM((ch, z_dim), jnp.float32),
            pltpu.SemaphoreType.DMA,
        ],
    )
    def embed_kernel(zemb_hbm, zf_hbm, emb_out, zf_v, rows_v, sem):
        wid = lax.axis_index("s") * nc + lax.axis_index("c")
        base = wid * bpw
        pltpu.sync_copy(zf_hbm.at[pl.ds(base, bpw)], zf_v)
        for c in range(nch):
            off = c * ch
            pltpu.async_copy(
                zemb_hbm.at[zf_v.at[pl.ds(off, ch)]], rows_v, sem).wait()
            pltpu.sync_copy(rows_v, emb_out.at[pl.ds(base + off, ch)])

    return embed_kernel(zemb, zf)


def _sc_gather(u, v, zemb, zf, pj, pk):
    n_nodes, atom_dim = u.shape
    z_dim = zemb.shape[1]
    pp = pj.shape[0]
    emb = _sc_embed(zemb, zf)
    info = plsc.get_sparse_core_info()
    nc, ns = info.num_cores, info.num_subcores
    nw = nc * ns
    bpw = pp // nw
    nch = bpw // _CH
    mesh = plsc.VectorSubcoreMesh(core_axis_name="c", subcore_axis_name="s")

    @functools.partial(
        pl.kernel,
        mesh=mesh,
        out_type=(
            jax.ShapeDtypeStruct((pp, atom_dim), jnp.float32),
            jax.ShapeDtypeStruct((pp, atom_dim), jnp.float32),
            jax.ShapeDtypeStruct((pp, z_dim), jnp.float32),
            jax.ShapeDtypeStruct((pp, z_dim), jnp.float32),
        ),
        scratch_types=[
            pltpu.VMEM((bpw,), jnp.int32),
            pltpu.VMEM((bpw,), jnp.int32),
            pltpu.VMEM((_CH, atom_dim), jnp.float32),
            pltpu.VMEM((_CH, atom_dim), jnp.float32),
            pltpu.VMEM((_CH, z_dim), jnp.float32),
            pltpu.VMEM((_CH, z_dim), jnp.float32),
            pltpu.SemaphoreType.DMA,
            pltpu.SemaphoreType.DMA,
            pltpu.SemaphoreType.DMA,
            pltpu.SemaphoreType.DMA,
        ],
    )
    def gather_kernel(u_hbm, v_hbm, emb_hbm, pj_hbm, pk_hbm,
                      hj_out, hk_out, ej_out, ek_out,
                      pj_v, pk_v, hj_v, hk_v, ej_v, ek_v,
                      sem_hj, sem_hk, sem_ej, sem_ek):
        wid = lax.axis_index("s") * nc + lax.axis_index("c")
        base = wid * bpw
        pltpu.sync_copy(pj_hbm.at[pl.ds(base, bpw)], pj_v)
        pltpu.sync_copy(pk_hbm.at[pl.ds(base, bpw)], pk_v)
        for c in range(nch):
            off = c * _CH
            cp_hj = pltpu.async_copy(u_hbm.at[pj_v.at[pl.ds(off, _CH)]], hj_v, sem_hj)
            cp_hk = pltpu.async_copy(v_hbm.at[pk_v.at[pl.ds(off, _CH)]], hk_v, sem_hk)
            cp_ej = pltpu.async_copy(emb_hbm.at[pj_v.at[pl.ds(off, _CH)]], ej_v, sem_ej)
            cp_ek = pltpu.async_copy(emb_hbm.at[pk_v.at[pl.ds(off, _CH)]], ek_v, sem_ek)
            cp_hj.wait()
            pltpu.sync_copy(hj_v, hj_out.at[pl.ds(base + off, _CH)])
            cp_hk.wait()
            pltpu.sync_copy(hk_v, hk_out.at[pl.ds(base + off, _CH)])
            cp_ej.wait()
            pltpu.sync_copy(ej_v, ej_out.at[pl.ds(base + off, _CH)])
            cp_ek.wait()
            pltpu.sync_copy(ek_v, ek_out.at[pl.ds(base + off, _CH)])

    return gather_kernel(u, v, emb, pj, pk)


def _prep_body(h_ref, wj_ref, wk_ref, u_ref, v_ref):
    u_ref[...] = jnp.dot(h_ref[...], wj_ref[...],
                         preferred_element_type=jnp.float32)
    v_ref[...] = jnp.dot(h_ref[...], wk_ref[...],
                         preferred_element_type=jnp.float32)


def _make_main_body(bp, b, n_en, s_dim, rbf_dim, cutoff, z_dim):
    f32 = jnp.float32
    width = cutoff / (rbf_dim - 1)

    def body(hj_ref, hk_ref, ej_ref, ek_ref, r0j_ref, r0k_ref, rjk_ref,
             ca_ref, seg_ref,
             wg1a_ref, wg1b_ref, wg1c_ref, wg1cos_ref,
             bg1_ref, wg2_ref, bg2_ref, wg3_ref, bg3_ref,
             ef_ref, w1pj_ref, w1pk_ref, w1pe_ref, b1p_ref,
             w2p_ref, b2p_ref, w3p_ref, b3p_ref,
             agg_ref, norm_ref):
        @pl.when(pl.program_id(0) == 0)
        def _init():
            agg_ref[...] = jnp.zeros_like(agg_ref)
            norm_ref[...] = jnp.zeros_like(norm_ref)

        cent = lax.broadcasted_iota(
            jnp.int32, (1, rbf_dim), 1).astype(f32) * width

        def dot(a, bb):
            return jnp.dot(a, bb, preferred_element_type=f32)

        def rbf(rc):
            rr = jnp.minimum(rc, cutoff)
            return jnp.exp(-0.5 * ((rr - cent) / width) ** 2)

        def ccut(rc):
            return 0.5 * (jnp.cos(jnp.pi * rc / cutoff) + 1.0) * (
                rc < cutoff).astype(f32)

        r0j = r0j_ref[...]
        r0k = r0k_ref[...]
        rjk = rjk_ref[...]
        x = (hj_ref[...]
             + hk_ref[...]
             + dot(rbf(r0j), wg1a_ref[...])
             + dot(rbf(r0k), wg1b_ref[...])
             + dot(rbf(rjk), wg1c_ref[...])
             + ca_ref[...] * wg1cos_ref[...]
             + bg1_ref[...])
        x = jax.nn.silu(x)
        x = jax.nn.silu(dot(x, wg2_ref[...]) + bg2_ref[...])
        gg = dot(x, wg3_ref[...]) + bg3_ref[...]

        cw = ccut(r0j) * ccut(r0k) * ccut(rjk)
        ohw = jnp.where(
            seg_ref[...] == lax.broadcasted_iota(jnp.int32, (bp, b), 1),
            cw, 0.0)

        epre = dot(ef_ref[...], w1pe_ref[...]) + b1p_ref[...]
        gp = (dot(ej_ref[...][:, :z_dim], w1pj_ref[...])
              + dot(ek_ref[...][:, :z_dim], w1pk_ref[...]))
        pre = jnp.concatenate(
            [gp + epre[e:e + 1, :] for e in range(n_en)], axis=0)
        h1 = jax.nn.silu(pre)
        h2 = jax.nn.silu(dot(h1, w2p_ref[...]) + b2p_ref[...])
        ge = dot(h2, w3p_ref[...]) + b3p_ref[...]

        tdims = (((0,), (0,)), ((), ()))
        for e in range(n_en):
            me = ge[e * bp:(e + 1) * bp, :] * gg
            agg_ref[:, e * s_dim:(e + 1) * s_dim] += lax.dot_general(
                ohw, me, tdims, preferred_element_type=f32)
        norm_ref[...] += jnp.sum(ohw, axis=0, keepdims=True)

    return body


def _out_body(agg_ref, nrm_ref, wo1_ref, bo1_ref, wo2_ref, bo2_ref, out_ref):
    x = agg_ref[...] / jnp.maximum(nrm_ref[...], 1e-8)
    h = jax.nn.silu(
        jnp.dot(x, wo1_ref[...], preferred_element_type=jnp.float32)
        + bo1_ref[...])
    out_ref[...] = (
        jnp.dot(h, wo2_ref[...], preferred_element_type=jnp.float32)
        + bo2_ref[...])


def _main_call(args, bp, b, n_en, s_dim, rbf_dim, z_dim, interp_shapes):
    pp, atom_dim = interp_shapes["hj"]
    ze_pad = interp_shapes["ej"][1]
    nblk = pp // bp
    row_spec = lambda d: pl.BlockSpec((bp, d), lambda i: (i, 0))
    full = lambda shape: pl.BlockSpec(shape, lambda i: (0, 0))
    in_specs = [
        row_spec(atom_dim), row_spec(atom_dim), row_spec(ze_pad),
        row_spec(ze_pad), row_spec(1), row_spec(1), row_spec(1), row_spec(1),
        row_spec(1),
    ] + [full(a.shape) for a in args[9:]]
    out_specs = [
        pl.BlockSpec((b, n_en * s_dim), lambda i: (0, 0)),
        pl.BlockSpec((1, b), lambda i: (0, 0)),
    ]
    out_shape = [
        jax.ShapeDtypeStruct((b, n_en * s_dim), jnp.float32),
        jax.ShapeDtypeStruct((1, b), jnp.float32),
    ]
    return pl.pallas_call(
        _make_main_body(bp, b, n_en, s_dim, rbf_dim, _CUTOFF, z_dim),
        grid=(nblk,),
        in_specs=in_specs,
        out_specs=out_specs,
        out_shape=out_shape,
    )(*args)


def kernel(h_flat, z_flat, e_feat, path_j, path_k, path_r0j, path_r0k,
           path_rjk, path_cosangle, path_batch, bsz, params):
    f32 = jnp.float32
    i32 = jnp.int32
    h_flat = h_flat.astype(f32)
    e_feat = e_feat.astype(f32)
    zemb = params["z_emb"].astype(f32)
    atom_dim = h_flat.shape[1]
    z_dim = zemb.shape[1]
    n_en = e_feat.shape[0]

    (g_w1, g_b1), (g_w2, g_b2), (g_w3, g_b3) = params["geom_mlp"]
    (p_w1, p_b1), (p_w2, p_b2), (p_w3, p_b3) = params["pair_mlp"]
    (o_w1, o_b1), (o_w2, o_b2) = params["out_mlp"]
    s_dim = g_w3.shape[1]
    rbf_dim = (g_w1.shape[0] - 2 * atom_dim - 1) // 3
    out_dim = o_w2.shape[1]

    p = path_j.shape[0]
    p_pad = ((p + 4095) // 4096) * 4096
    pad = p_pad - p
    pj = jnp.pad(path_j.astype(i32), (0, pad))
    pk = jnp.pad(path_k.astype(i32), (0, pad))
    seg = jnp.pad(jnp.minimum(path_batch, bsz - 1).astype(i32), (0, pad))
    r0j = jnp.pad(path_r0j.astype(f32), (0, pad), constant_values=1e9)
    r0k = jnp.pad(path_r0k.astype(f32), (0, pad), constant_values=1e9)
    rjk = jnp.pad(path_rjk.astype(f32), (0, pad), constant_values=1e9)
    ca = jnp.pad(path_cosangle.astype(f32), (0, pad))

    zemb_pad = jnp.pad(zemb, ((0, 0), (0, 128 - z_dim)))
    g_hidden = g_w2.shape[0]
    wg1hj_full = g_w1[0:atom_dim]
    wg1hk_full = g_w1[atom_dim:2 * atom_dim]
    u, v = pl.pallas_call(
        _prep_body,
        out_shape=[
            jax.ShapeDtypeStruct((h_flat.shape[0], g_hidden), jnp.float32),
            jax.ShapeDtypeStruct((h_flat.shape[0], g_hidden), jnp.float32),
        ],
    )(h_flat, wg1hj_full, wg1hk_full)
    hj, hk, ej, ek = _sc_gather(u, v, zemb_pad, z_flat.astype(i32), pj, pk)

    a = atom_dim
    r = rbf_dim
    wg1a = g_w1[2 * a:2 * a + r]
    wg1b = g_w1[2 * a + r:2 * a + 2 * r]
    wg1c = g_w1[2 * a + 2 * r:2 * a + 3 * r]
    wg1cos = g_w1[2 * a + 3 * r:2 * a + 3 * r + 1]
    w1pj = p_w1[0:z_dim]
    w1pk = p_w1[z_dim:2 * z_dim]
    w1pe = p_w1[2 * z_dim:2 * z_dim + e_feat.shape[1]]

    args = (
        hj, hk, ej, ek,
        r0j.reshape(p_pad, 1), r0k.reshape(p_pad, 1), rjk.reshape(p_pad, 1),
        ca.reshape(p_pad, 1), seg.reshape(p_pad, 1),
        wg1a, wg1b, wg1c, wg1cos, g_b1.reshape(1, -1),
        g_w2, g_b2.reshape(1, -1), g_w3, g_b3.reshape(1, -1),
        e_feat, w1pj, w1pk, w1pe, p_b1.reshape(1, -1),
        p_w2, p_b2.reshape(1, -1), p_w3, p_b3.reshape(1, -1),
    )
    agg, normv = _main_call(
        args, _BP, _B, n_en, s_dim, rbf_dim, z_dim,
        {"hj": hj.shape, "ej": ej.shape})

    aggr = agg.reshape(_B, n_en, s_dim).reshape(_B * n_en, s_dim)
    nrep = jnp.broadcast_to(
        normv.reshape(_B, 1), (_B, n_en)).reshape(_B * n_en, 1)
    out2 = pl.pallas_call(
        _out_body,
        out_shape=jax.ShapeDtypeStruct((_B * n_en, out_dim), jnp.float32),
    )(aggr, nrep, o_w1, o_b1.reshape(1, -1), o_w2, o_b2.reshape(1, -1))
    return out2.reshape(_B, n_en, out_dim)
